# baseline (speedup 1.0000x reference)
import functools

import jax
import jax.numpy as jnp
from jax import lax
from jax.experimental import pallas as pl
from jax.experimental.pallas import tpu as pltpu
from jax.experimental.pallas import tpu_sc as plsc

DIM = 64
NC = 2
NS = 16
NW = NC * NS

GATHER = 128
GROUPS = 10
CHUNK = GATHER * GROUPS


def _make_gather(b_total: int):
  b_per_w = b_total // NW
  n_chunks = b_per_w // CHUNK
  n_groups_w = b_per_w // GATHER
  mesh = plsc.VectorSubcoreMesh(core_axis_name="c", subcore_axis_name="s")

  @functools.partial(
      pl.kernel,
      mesh=mesh,
      out_type=jax.ShapeDtypeStruct((b_total, DIM), jnp.float32),
      compiler_params=pltpu.CompilerParams(use_tc_tiling_on_sc=False),
      scratch_types=[
          pltpu.VMEM((n_groups_w, GATHER), jnp.int32),
          pltpu.VMEM((CHUNK, DIM), jnp.float32),
          pltpu.SemaphoreType.DMA,
      ],
  )
  def gather_kernel(table_hbm, idx_hbm, out_hbm, idx_v, rows_v, sem):
    wid = lax.axis_index("s") * NC + lax.axis_index("c")
    base = wid * b_per_w
    pltpu.sync_copy(idx_hbm.at[wid], idx_v)

    def body(c, carry):
      for g in range(GROUPS):
        pltpu.async_copy(
            table_hbm.at[idx_v.at[c * GROUPS + g]],
            rows_v.at[pl.ds(g * GATHER, GATHER)],
            sem,
        )
      for g in range(GROUPS):
        pltpu.make_async_copy(
            table_hbm.at[idx_v.at[0]],
            rows_v.at[pl.ds(g * GATHER, GATHER)],
            sem,
        ).wait()
      pltpu.sync_copy(rows_v, out_hbm.at[pl.ds(base + c * CHUNK, CHUNK)])
      return carry

    lax.fori_loop(0, n_chunks, body, None)

  return gather_kernel


def kernel(token_ids, embeddings):
  b, t = token_ids.shape
  b_total = b * t
  idx = token_ids.reshape(NW, b_total // (NW * GATHER), GATHER).astype(jnp.int32)
  out = _make_gather(b_total)(embeddings, idx)
  return out.reshape(b, t, DIM)


# speedup vs baseline: 1.0088x; 1.0088x over previous
import functools

import jax
import jax.numpy as jnp
from jax import lax
from jax.experimental import pallas as pl
from jax.experimental.pallas import tpu as pltpu
from jax.experimental.pallas import tpu_sc as plsc

DIM = 64
NC = 2
NS = 16
NW = NC * NS

GATHER = 128
GROUPS = 5
CHUNK = GATHER * GROUPS


def _make_gather(b_total: int):
  b_per_w = b_total // NW
  n_chunks = b_per_w // CHUNK
  n_groups_w = b_per_w // GATHER
  mesh = plsc.VectorSubcoreMesh(core_axis_name="c", subcore_axis_name="s")

  @functools.partial(
      pl.kernel,
      mesh=mesh,
      out_type=jax.ShapeDtypeStruct((b_total, DIM), jnp.float32),
      compiler_params=pltpu.CompilerParams(use_tc_tiling_on_sc=False),
      scratch_types=[
          pltpu.VMEM((n_groups_w, GATHER), jnp.int32),
          pltpu.VMEM((2, CHUNK, DIM), jnp.float32),
          pltpu.SemaphoreType.DMA,
          pltpu.SemaphoreType.DMA,
          pltpu.SemaphoreType.DMA,
      ],
  )
  def gather_kernel(table_hbm, idx_hbm, out_hbm, idx_v, rows_v, gsem0, gsem1,
                    osem):
    wid = lax.axis_index("s") * NC + lax.axis_index("c")
    base = wid * b_per_w
    gsems = (gsem0, gsem1)
    pltpu.sync_copy(idx_hbm.at[wid], idx_v)

    def fire(c, b):
      for g in range(GROUPS):
        pltpu.async_copy(
            table_hbm.at[idx_v.at[c * GROUPS + g]],
            rows_v.at[b, pl.ds(g * GATHER, GATHER)],
            gsems[b],
        )

    fire(0, 0)

    def pair_body(p, carry):
      for b in range(2):
        c = 2 * p + b
        buf = rows_v.at[b]

        @pl.when(c >= 1)
        def _wait_prev_out():
          pltpu.make_async_copy(
              rows_v.at[1 - b], out_hbm.at[pl.ds(0, CHUNK)], osem).wait()

        @pl.when(c + 1 < n_chunks)
        def _fire_next():
          fire(c + 1, 1 - b)

        pltpu.make_async_copy(
            out_hbm.at[pl.ds(0, CHUNK)], buf, gsems[b]).wait()
        pltpu.async_copy(buf, out_hbm.at[pl.ds(base + c * CHUNK, CHUNK)], osem)
      return carry

    lax.fori_loop(0, n_chunks // 2, pair_body, None)
    pltpu.make_async_copy(
        rows_v.at[1], out_hbm.at[pl.ds(0, CHUNK)], osem).wait()

  return gather_kernel


def kernel(token_ids, embeddings):
  b, t = token_ids.shape
  b_total = b * t
  idx = token_ids.reshape(NW, b_total // (NW * GATHER), GATHER).astype(jnp.int32)
  out = _make_gather(b_total)(embeddings, idx)
  return out.reshape(b, t, DIM)
